# baseline (speedup 1.0000x reference)
import functools

import jax
import jax.numpy as jnp
from jax import lax
from jax.experimental import pallas as pl
from jax.experimental.pallas import tpu as pltpu
from jax.experimental.pallas import tpu_sc as plsc

M = 16384
NNZ = 268435
N = 4
FIN = 64
K = 3
OUT = 64
F = N * FIN
NLANE = 16

NW = 32
ROWS_PER_W = M // NW
SUB = 64
NSUB = ROWS_PER_W // SUB
NBLK = M // SUB
EC = 64
NNZ_PAD = ((NNZ + EC - 1) // EC) * EC
BOUNDS_PAD = NBLK + NLANE

BM = 1024


def _spmm_sc_body(v_hbm, vals_hbm, cols_hbm, rows_hbm, bounds_hbm, out_hbm,
                  bounds_v, rows_v, cols_v, vals_v, gbuf, acc, sem):
    wid = lax.axis_index("s") * 2 + lax.axis_index("c")
    lane = lax.broadcasted_iota(jnp.int32, (NLANE,), 0)
    flane = [(lane + r) & (NLANE - 1) for r in range(NLANE)]

    pltpu.sync_copy(bounds_hbm.at[pl.ds(wid * NSUB, NLANE)], bounds_v)
    bvec = bounds_v[...]

    bvals = [bvec[i] for i in range(NSUB + 1)]

    def sub_block(b, carry):
        sub_base = wid * ROWS_PER_W + b * SUB
        s_e = bvals[0]
        e_e = bvals[1]
        for i in range(1, NSUB + 1):
            if i < NSUB:
                s_e = jnp.where(b == i, bvals[i], s_e)
            e_e = jnp.where(b + 1 == i, bvals[i], e_e)
        c0 = s_e // EC
        c1 = (e_e + EC - 1) // EC

        def zero_row(r, carry2):
            z = jnp.zeros((NLANE,), jnp.float32)
            for ff in range(F // NLANE):
                acc[r, pl.ds(ff * NLANE, NLANE)] = z
            return carry2

        lax.fori_loop(0, SUB, zero_row, 0)

        def chunk(c, carry2):
            e0 = c * EC
            pltpu.sync_copy(rows_hbm.at[pl.ds(e0, EC)], rows_v)
            pltpu.sync_copy(cols_hbm.at[pl.ds(e0, EC)], cols_v)
            pltpu.sync_copy(vals_hbm.at[pl.ds(e0, EC)], vals_v)
            pltpu.async_copy(v_hbm.at[cols_v], gbuf, sem).wait()

            def group(g, carry3):
                rvec = rows_v[pl.ds(g * NLANE, NLANE)] - sub_base
                ok = (rvec >= 0) & (rvec < SUB)
                rloc = jnp.where(ok, rvec, SUB)
                vvec = vals_v[pl.ds(g * NLANE, NLANE)]
                evec = g * NLANE + lane
                for b16 in range(F // NLANE):
                    for r in range(NLANE):
                        f_vec = flane[r] + (b16 * NLANE)
                        x = plsc.load_gather(gbuf, [evec, f_vec])
                        plsc.addupdate_scatter(acc, [rloc, f_vec], x * vvec)
                return carry3

            lax.fori_loop(0, EC // NLANE, group, 0)
            return carry2

        lax.fori_loop(c0, c1, chunk, 0)
        pltpu.sync_copy(acc.at[pl.ds(0, SUB)], out_hbm.at[pl.ds(sub_base, SUB)])
        return carry

    lax.fori_loop(0, NSUB, sub_block, 0)


def _spmm_sc(v, vals_p, cols_p, rows_p, bounds):
    mesh = plsc.VectorSubcoreMesh(core_axis_name="c", subcore_axis_name="s")
    fn = pl.kernel(
        _spmm_sc_body,
        mesh=mesh,
        compiler_params=pltpu.CompilerParams(use_tc_tiling_on_sc=False,
                                             needs_layout_passes=False),
        out_type=jax.ShapeDtypeStruct((M, F), jnp.float32),
        scratch_types=[
            pltpu.VMEM((NLANE,), jnp.int32),
            pltpu.VMEM((EC,), jnp.int32),
            pltpu.VMEM((EC,), jnp.int32),
            pltpu.VMEM((EC,), jnp.float32),
            pltpu.VMEM((EC, F), jnp.float32),
            pltpu.VMEM((SUB + 1, F), jnp.float32),
            pltpu.SemaphoreType.DMA,
        ],
    )
    return fn(v, vals_p, cols_p, rows_p, bounds)


def _gemm_body(x0_ref, x1_ref, y2_ref, w_ref, b_ref, o_ref):
    xcat = jnp.concatenate([x0_ref[...], x1_ref[...], y2_ref[...]], axis=1)
    z = lax.dot_general(
        xcat, w_ref[...], (((1,), (0,)), ((), ())),
        preferred_element_type=jnp.float32,
        precision=lax.Precision.HIGHEST,
    ) + b_ref[...]
    o_ref[...] = jnp.where(z > 0, z, jnp.exp(jnp.minimum(z, 0.0)) - 1.0)


def _gemm_elu(x0, x1, y2, wbd, bias_t):
    return pl.pallas_call(
        _gemm_body,
        grid=(M // BM,),
        in_specs=[
            pl.BlockSpec((BM, F), lambda i: (i, 0)),
            pl.BlockSpec((BM, F), lambda i: (i, 0)),
            pl.BlockSpec((BM, F), lambda i: (i, 0)),
            pl.BlockSpec((3 * F, N * OUT), lambda i: (0, 0)),
            pl.BlockSpec((1, N * OUT), lambda i: (0, 0)),
        ],
        out_specs=pl.BlockSpec((BM, N * OUT), lambda i: (i, 0)),
        out_shape=jax.ShapeDtypeStruct((M, N * OUT), jnp.float32),
    )(x0, x1, y2, wbd, bias_t)


def kernel(x, L_values, L_rows, L_cols, weight, bias):
    rows = L_rows.astype(jnp.int32)
    cols = L_cols.astype(jnp.int32)

    x0 = jnp.transpose(x, (1, 0, 2)).reshape(M, F)

    pad = NNZ_PAD - NNZ
    rows_p = jnp.concatenate([rows, jnp.full((pad,), M, jnp.int32)])
    cols_p = jnp.concatenate([cols, jnp.zeros((pad,), jnp.int32)])
    vals_p = jnp.concatenate([L_values, jnp.zeros((pad,), jnp.float32)])

    bounds = jnp.searchsorted(rows, jnp.arange(NBLK + 1, dtype=jnp.int32) * SUB).astype(jnp.int32)
    bounds = jnp.concatenate([bounds, jnp.full((BOUNDS_PAD - NBLK - 1,), NNZ, jnp.int32)])

    x1 = _spmm_sc(x0, vals_p, cols_p, rows_p, bounds)
    y2 = _spmm_sc(x1, vals_p, cols_p, rows_p, bounds)

    w = weight.reshape(FIN, K, OUT)
    eye_n = jnp.eye(N, dtype=jnp.float32)

    def bd(wk):
        return (eye_n[:, None, :, None] * wk[None, :, None, :]).reshape(N * FIN, N * OUT)

    wbd = jnp.concatenate([bd(w[:, 0] - w[:, 2]), bd(w[:, 1]), bd(2.0 * w[:, 2])], axis=0)
    bias_t = jnp.tile(bias, (N,)).reshape(1, N * OUT)

    out = _gemm_elu(x0, x1, y2, wbd, bias_t)
    return out.reshape(M, N, OUT).transpose(1, 0, 2)


# speedup vs baseline: 1.5563x; 1.5563x over previous
import functools

import jax
import jax.numpy as jnp
from jax import lax
from jax.experimental import pallas as pl
from jax.experimental.pallas import tpu as pltpu
from jax.experimental.pallas import tpu_sc as plsc

M = 16384
NNZ = 268435
N = 4
FIN = 64
K = 3
OUT = 64
F = N * FIN
NLANE = 16

NW = 32
ROWS_PER_W = M // NW
SUB = 128
NSUB = ROWS_PER_W // SUB
NBLK = M // SUB
EC = 128
MDEPTH = 4
NNZ_PAD = ((NNZ + EC - 1) // EC) * EC
BOUNDS_PAD = NBLK + NLANE

BM = 1024


def _spmm_sc_body(v_hbm, vals_hbm, cols_hbm, rows_hbm, bounds_hbm, out_hbm,
                  bounds_v, rows_v, cols_v, vals_v, gbuf, acc, sem_m, sem_g):
    wid = lax.axis_index("s") * 2 + lax.axis_index("c")
    lane = lax.broadcasted_iota(jnp.int32, (NLANE,), 0)
    flane = [(lane + r) & (NLANE - 1) for r in range(NLANE)]

    pltpu.sync_copy(bounds_hbm.at[pl.ds(wid * 8, NLANE)], bounds_v)
    bvec = bounds_v[...]

    bvals = [bvec[i] for i in range(NSUB + 1)]

    def sub_block(b, carry):
        sub_base = wid * ROWS_PER_W + b * SUB
        s_e = bvals[0]
        e_e = bvals[1]
        for i in range(1, NSUB + 1):
            if i < NSUB:
                s_e = jnp.where(b == i, bvals[i], s_e)
            e_e = jnp.where(b + 1 == i, bvals[i], e_e)
        c0 = s_e // EC
        c1 = (e_e + EC - 1) // EC

        def zero_row(r, carry2):
            z = jnp.zeros((NLANE,), jnp.float32)
            for ff in range(F // NLANE):
                acc[r, pl.ds(ff * NLANE, NLANE)] = z
            return carry2

        lax.fori_loop(0, SUB, zero_row, 0)

        def fire_meta(c):
            e0 = c * EC
            m0 = (c & (MDEPTH - 1)) * EC
            pltpu.async_copy(rows_hbm.at[pl.ds(e0, EC)], rows_v.at[pl.ds(m0, EC)], sem_m)
            pltpu.async_copy(cols_hbm.at[pl.ds(e0, EC)], cols_v.at[pl.ds(m0, EC)], sem_m)
            pltpu.async_copy(vals_hbm.at[pl.ds(e0, EC)], vals_v.at[pl.ds(m0, EC)], sem_m)

        def wait_meta():
            pltpu.make_async_copy(rows_hbm.at[pl.ds(0, EC)], rows_v.at[pl.ds(0, EC)], sem_m).wait()
            pltpu.make_async_copy(cols_hbm.at[pl.ds(0, EC)], cols_v.at[pl.ds(0, EC)], sem_m).wait()
            pltpu.make_async_copy(vals_hbm.at[pl.ds(0, EC)], vals_v.at[pl.ds(0, EC)], sem_m).wait()

        def fire_gather(c):
            m0 = (c & (MDEPTH - 1)) * EC
            g0 = (c & 1) * EC
            pltpu.async_copy(v_hbm.at[cols_v.at[pl.ds(m0, EC)]],
                             gbuf.at[pl.ds(g0, EC)], sem_g)

        def wait_gather():
            pltpu.make_async_copy(v_hbm.at[pl.ds(0, EC)], gbuf.at[pl.ds(0, EC)], sem_g).wait()

        @pl.when(c0 < c1)
        def _():
            fire_meta(c0)

            @pl.when(c0 + 1 < c1)
            def _():
                fire_meta(c0 + 1)

            wait_meta()
            fire_gather(c0)

        def chunk(c, carry2):
            wait_gather()

            @pl.when(c + 2 < c1)
            def _():
                fire_meta(c + 2)

            @pl.when(c + 1 < c1)
            def _():
                wait_meta()
                fire_gather(c + 1)

            m0 = (c & (MDEPTH - 1)) * EC
            g0 = (c & 1) * EC

            def group(g, carry3):
                rvec = rows_v[pl.ds(m0 + g * NLANE, NLANE)] - sub_base
                ok = (rvec >= 0) & (rvec < SUB)
                rloc = jnp.where(ok, rvec, SUB)
                vvec = vals_v[pl.ds(m0 + g * NLANE, NLANE)]
                evec = g0 + g * NLANE + lane
                for b16 in range(F // NLANE):
                    for r in range(NLANE):
                        f_vec = flane[r] + (b16 * NLANE)
                        x = plsc.load_gather(gbuf, [evec, f_vec])
                        plsc.addupdate_scatter(acc, [rloc, f_vec], x * vvec)
                return carry3

            lax.fori_loop(0, EC // NLANE, group, 0)
            return carry2

        lax.fori_loop(c0, c1, chunk, 0)
        pltpu.sync_copy(acc.at[pl.ds(0, SUB)], out_hbm.at[pl.ds(sub_base, SUB)])
        return carry

    lax.fori_loop(0, NSUB, sub_block, 0)


def _spmm_sc(v, vals_p, cols_p, rows_p, bounds):
    mesh = plsc.VectorSubcoreMesh(core_axis_name="c", subcore_axis_name="s")
    fn = pl.kernel(
        _spmm_sc_body,
        mesh=mesh,
        compiler_params=pltpu.CompilerParams(use_tc_tiling_on_sc=False,
                                             needs_layout_passes=False),
        out_type=jax.ShapeDtypeStruct((M, F), jnp.float32),
        scratch_types=[
            pltpu.VMEM((NLANE,), jnp.int32),
            pltpu.VMEM((MDEPTH * EC,), jnp.int32),
            pltpu.VMEM((MDEPTH * EC,), jnp.int32),
            pltpu.VMEM((MDEPTH * EC,), jnp.float32),
            pltpu.VMEM((2 * EC, F), jnp.float32),
            pltpu.VMEM((SUB + 1, F), jnp.float32),
            pltpu.SemaphoreType.DMA,
            pltpu.SemaphoreType.DMA,
        ],
    )
    return fn(v, vals_p, cols_p, rows_p, bounds)


def _gemm_body(x0_ref, x1_ref, y2_ref, w_ref, b_ref, o_ref):
    xcat = jnp.concatenate([x0_ref[...], x1_ref[...], y2_ref[...]], axis=1)
    z = lax.dot_general(
        xcat, w_ref[...], (((1,), (0,)), ((), ())),
        preferred_element_type=jnp.float32,
        precision=lax.Precision.HIGHEST,
    ) + b_ref[...]
    o_ref[...] = jnp.where(z > 0, z, jnp.exp(jnp.minimum(z, 0.0)) - 1.0)


def _gemm_elu(x0, x1, y2, wbd, bias_t):
    return pl.pallas_call(
        _gemm_body,
        grid=(M // BM,),
        in_specs=[
            pl.BlockSpec((BM, F), lambda i: (i, 0)),
            pl.BlockSpec((BM, F), lambda i: (i, 0)),
            pl.BlockSpec((BM, F), lambda i: (i, 0)),
            pl.BlockSpec((3 * F, N * OUT), lambda i: (0, 0)),
            pl.BlockSpec((1, N * OUT), lambda i: (0, 0)),
        ],
        out_specs=pl.BlockSpec((BM, N * OUT), lambda i: (i, 0)),
        out_shape=jax.ShapeDtypeStruct((M, N * OUT), jnp.float32),
    )(x0, x1, y2, wbd, bias_t)


def kernel(x, L_values, L_rows, L_cols, weight, bias):
    rows = L_rows.astype(jnp.int32)
    cols = L_cols.astype(jnp.int32)

    x0 = jnp.transpose(x, (1, 0, 2)).reshape(M, F)

    pad = NNZ_PAD - NNZ
    rows_p = jnp.concatenate([rows, jnp.full((pad,), M, jnp.int32)])
    cols_p = jnp.concatenate([cols, jnp.zeros((pad,), jnp.int32)])
    vals_p = jnp.concatenate([L_values, jnp.zeros((pad,), jnp.float32)])

    bounds = jnp.searchsorted(rows, jnp.arange(NBLK + 1, dtype=jnp.int32) * SUB).astype(jnp.int32)
    widx = (jnp.arange(NW)[:, None] * NSUB + jnp.arange(8)[None, :]).clip(0, NBLK)
    bounds = jnp.concatenate([bounds[widx].reshape(-1), jnp.full((NLANE,), NNZ, jnp.int32)])

    x1 = _spmm_sc(x0, vals_p, cols_p, rows_p, bounds)
    y2 = _spmm_sc(x1, vals_p, cols_p, rows_p, bounds)

    w = weight.reshape(FIN, K, OUT)
    eye_n = jnp.eye(N, dtype=jnp.float32)

    def bd(wk):
        return (eye_n[:, None, :, None] * wk[None, :, None, :]).reshape(N * FIN, N * OUT)

    wbd = jnp.concatenate([bd(w[:, 0] - w[:, 2]), bd(w[:, 1]), bd(2.0 * w[:, 2])], axis=0)
    bias_t = jnp.tile(bias, (N,)).reshape(1, N * OUT)

    out = _gemm_elu(x0, x1, y2, wbd, bias_t)
    return out.reshape(M, N, OUT).transpose(1, 0, 2)


# speedup vs baseline: 5.1383x; 3.3017x over previous
import functools

import jax
import jax.numpy as jnp
from jax import lax
from jax.experimental import pallas as pl
from jax.experimental.pallas import tpu as pltpu
from jax.experimental.pallas import tpu_sc as plsc

M = 16384
NNZ = 268435
N = 4
FIN = 64
K = 3
OUT = 64
F = N * FIN
NLANE = 16

NW = 32
ROWS_PER_W = M // NW
SUB = 128
NSUB = ROWS_PER_W // SUB
NBLK = M // SUB
EC = 128
MDEPTH = 4
NNZ_PAD = ((NNZ + EC - 1) // EC) * EC
BOUNDS_PAD = NBLK + NLANE

BM = 1024


def _spmm_sc_body(v_hbm, vals_hbm, cols_hbm, rows_hbm, bounds_hbm, out_hbm,
                  bounds_v, rows_v, cols_v, vals_v, gbuf, acc, sem_m, sem_g):
    wid = lax.axis_index("s") * 2 + lax.axis_index("c")
    lane = lax.broadcasted_iota(jnp.int32, (NLANE,), 0)
    flane = [(lane + r) & (NLANE - 1) for r in range(NLANE)]

    pltpu.sync_copy(bounds_hbm.at[pl.ds(wid * 8, NLANE)], bounds_v)
    bvec = bounds_v[...]

    bvals = [bvec[i] for i in range(NSUB + 1)]

    def sub_block(b, carry):
        sub_base = wid * ROWS_PER_W + b * SUB
        s_e = bvals[0]
        e_e = bvals[1]
        for i in range(1, NSUB + 1):
            if i < NSUB:
                s_e = jnp.where(b == i, bvals[i], s_e)
            e_e = jnp.where(b + 1 == i, bvals[i], e_e)
        c0 = s_e // EC
        c1 = (e_e + EC - 1) // EC

        def zero_row(r, carry2):
            z = jnp.zeros((NLANE,), jnp.float32)
            for ff in range(F // NLANE):
                acc[r, pl.ds(ff * NLANE, NLANE)] = z
            return carry2

        lax.fori_loop(0, SUB, zero_row, 0)

        def fire_meta(c):
            e0 = c * EC
            m0 = (c & (MDEPTH - 1)) * EC
            pltpu.async_copy(rows_hbm.at[pl.ds(e0, EC)], rows_v.at[pl.ds(m0, EC)], sem_m)
            pltpu.async_copy(cols_hbm.at[pl.ds(e0, EC)], cols_v.at[pl.ds(m0, EC)], sem_m)
            pltpu.async_copy(vals_hbm.at[pl.ds(e0, EC)], vals_v.at[pl.ds(m0, EC)], sem_m)

        def wait_meta():
            pltpu.make_async_copy(rows_hbm.at[pl.ds(0, EC)], rows_v.at[pl.ds(0, EC)], sem_m).wait()
            pltpu.make_async_copy(cols_hbm.at[pl.ds(0, EC)], cols_v.at[pl.ds(0, EC)], sem_m).wait()
            pltpu.make_async_copy(vals_hbm.at[pl.ds(0, EC)], vals_v.at[pl.ds(0, EC)], sem_m).wait()

        def fire_gather(c):
            m0 = (c & (MDEPTH - 1)) * EC
            g0 = (c & 1) * EC
            pltpu.async_copy(v_hbm.at[cols_v.at[pl.ds(m0, EC)]],
                             gbuf.at[pl.ds(g0, EC)], sem_g)

        def wait_gather():
            pltpu.make_async_copy(v_hbm.at[pl.ds(0, EC)], gbuf.at[pl.ds(0, EC)], sem_g).wait()

        @pl.when(c0 < c1)
        def _():
            fire_meta(c0)

            @pl.when(c0 + 1 < c1)
            def _():
                fire_meta(c0 + 1)

            wait_meta()
            fire_gather(c0)

        def chunk(c, carry2):
            wait_gather()

            @pl.when(c + 2 < c1)
            def _():
                fire_meta(c + 2)

            @pl.when(c + 1 < c1)
            def _():
                wait_meta()
                fire_gather(c + 1)

            m0 = (c & (MDEPTH - 1)) * EC
            g0 = (c & 1) * EC

            def group(g, carry3):
                rvec = rows_v[pl.ds(m0 + g * NLANE, NLANE)] - sub_base
                ok = (rvec >= 0) & (rvec < SUB)
                rloc = jnp.where(ok, rvec, SUB)
                vvec = vals_v[pl.ds(m0 + g * NLANE, NLANE)]
                rows_j = [rloc[j] for j in range(NLANE)]
                vals_j = [vvec[j] for j in range(NLANE)]
                for ff in range(F // NLANE):
                    fs = pl.ds(ff * NLANE, NLANE)
                    xs = [gbuf[g0 + g * NLANE + j, fs] for j in range(NLANE)]
                    ys = [vals_j[j] * xs[j] for j in range(NLANE)]
                    for j in range(NLANE):
                        plsc.addupdate(acc.at[rows_j[j], fs], ys[j])
                return carry3

            lax.fori_loop(0, EC // NLANE, group, 0)
            return carry2

        lax.fori_loop(c0, c1, chunk, 0)
        pltpu.sync_copy(acc.at[pl.ds(0, SUB)], out_hbm.at[pl.ds(sub_base, SUB)])
        return carry

    lax.fori_loop(0, NSUB, sub_block, 0)


def _spmm_sc(v, vals_p, cols_p, rows_p, bounds):
    mesh = plsc.VectorSubcoreMesh(core_axis_name="c", subcore_axis_name="s")
    fn = pl.kernel(
        _spmm_sc_body,
        mesh=mesh,
        compiler_params=pltpu.CompilerParams(use_tc_tiling_on_sc=False,
                                             needs_layout_passes=False),
        out_type=jax.ShapeDtypeStruct((M, F), jnp.float32),
        scratch_types=[
            pltpu.VMEM((NLANE,), jnp.int32),
            pltpu.VMEM((MDEPTH * EC,), jnp.int32),
            pltpu.VMEM((MDEPTH * EC,), jnp.int32),
            pltpu.VMEM((MDEPTH * EC,), jnp.float32),
            pltpu.VMEM((2 * EC, F), jnp.float32),
            pltpu.VMEM((SUB + 1, F), jnp.float32),
            pltpu.SemaphoreType.DMA,
            pltpu.SemaphoreType.DMA,
        ],
    )
    return fn(v, vals_p, cols_p, rows_p, bounds)


def _gemm_body(x0_ref, x1_ref, y2_ref, w_ref, b_ref, o_ref):
    xcat = jnp.concatenate([x0_ref[...], x1_ref[...], y2_ref[...]], axis=1)
    z = lax.dot_general(
        xcat, w_ref[...], (((1,), (0,)), ((), ())),
        preferred_element_type=jnp.float32,
        precision=lax.Precision.HIGHEST,
    ) + b_ref[...]
    o_ref[...] = jnp.where(z > 0, z, jnp.exp(jnp.minimum(z, 0.0)) - 1.0)


def _gemm_elu(x0, x1, y2, wbd, bias_t):
    return pl.pallas_call(
        _gemm_body,
        grid=(M // BM,),
        in_specs=[
            pl.BlockSpec((BM, F), lambda i: (i, 0)),
            pl.BlockSpec((BM, F), lambda i: (i, 0)),
            pl.BlockSpec((BM, F), lambda i: (i, 0)),
            pl.BlockSpec((3 * F, N * OUT), lambda i: (0, 0)),
            pl.BlockSpec((1, N * OUT), lambda i: (0, 0)),
        ],
        out_specs=pl.BlockSpec((BM, N * OUT), lambda i: (i, 0)),
        out_shape=jax.ShapeDtypeStruct((M, N * OUT), jnp.float32),
    )(x0, x1, y2, wbd, bias_t)


def kernel(x, L_values, L_rows, L_cols, weight, bias):
    rows = L_rows.astype(jnp.int32)
    cols = L_cols.astype(jnp.int32)

    x0 = jnp.transpose(x, (1, 0, 2)).reshape(M, F)

    pad = NNZ_PAD - NNZ
    rows_p = jnp.concatenate([rows, jnp.full((pad,), M, jnp.int32)])
    cols_p = jnp.concatenate([cols, jnp.zeros((pad,), jnp.int32)])
    vals_p = jnp.concatenate([L_values, jnp.zeros((pad,), jnp.float32)])

    bounds = jnp.searchsorted(rows, jnp.arange(NBLK + 1, dtype=jnp.int32) * SUB).astype(jnp.int32)
    widx = (jnp.arange(NW)[:, None] * NSUB + jnp.arange(8)[None, :]).clip(0, NBLK)
    bounds = jnp.concatenate([bounds[widx].reshape(-1), jnp.full((NLANE,), NNZ, jnp.int32)])

    x1 = _spmm_sc(x0, vals_p, cols_p, rows_p, bounds)
    y2 = _spmm_sc(x1, vals_p, cols_p, rows_p, bounds)

    w = weight.reshape(FIN, K, OUT)
    eye_n = jnp.eye(N, dtype=jnp.float32)

    def bd(wk):
        return (eye_n[:, None, :, None] * wk[None, :, None, :]).reshape(N * FIN, N * OUT)

    wbd = jnp.concatenate([bd(w[:, 0] - w[:, 2]), bd(w[:, 1]), bd(2.0 * w[:, 2])], axis=0)
    bias_t = jnp.tile(bias, (N,)).reshape(1, N * OUT)

    out = _gemm_elu(x0, x1, y2, wbd, bias_t)
    return out.reshape(M, N, OUT).transpose(1, 0, 2)


# speedup vs baseline: 5.6489x; 1.0994x over previous
import functools

import jax
import jax.numpy as jnp
from jax import lax
from jax.experimental import pallas as pl
from jax.experimental.pallas import tpu as pltpu
from jax.experimental.pallas import tpu_sc as plsc

M = 16384
NNZ = 268435
N = 4
FIN = 64
K = 3
OUT = 64
F = N * FIN
NLANE = 16

NW = 32
ROWS_PER_W = M // NW
SUB = 128
NSUB = ROWS_PER_W // SUB
NBLK = M // SUB
EC = 128
MDEPTH = 4
NNZ_PAD = ((NNZ + EC - 1) // EC) * EC
BOUNDS_PAD = NBLK + NLANE

BM = 1024


def _spmm_sc_body(v_hbm, vals_hbm, cols_hbm, rows_hbm, bounds_hbm, out_hbm,
                  bounds_v, rows_v, cols_v, vals_v, gbuf, acc, sem_m, sem_g):
    wid = lax.axis_index("s") * 2 + lax.axis_index("c")
    lane = lax.broadcasted_iota(jnp.int32, (NLANE,), 0)
    flane = [(lane + r) & (NLANE - 1) for r in range(NLANE)]

    pltpu.sync_copy(bounds_hbm.at[pl.ds(wid * 8, NLANE)], bounds_v)
    bvec = bounds_v[...]

    bvals = [bvec[i] for i in range(NSUB + 1)]

    def sub_block(b, carry):
        sub_base = wid * ROWS_PER_W + b * SUB
        s_e = bvals[0]
        e_e = bvals[1]
        for i in range(1, NSUB + 1):
            if i < NSUB:
                s_e = jnp.where(b == i, bvals[i], s_e)
            e_e = jnp.where(b + 1 == i, bvals[i], e_e)
        c0 = s_e // EC
        c1 = (e_e + EC - 1) // EC

        def zero_row(r, carry2):
            z = jnp.zeros((NLANE,), jnp.float32)
            for ff in range(F // NLANE):
                acc[r, pl.ds(ff * NLANE, NLANE)] = z
            return carry2

        lax.fori_loop(0, SUB, zero_row, 0)

        def fire_meta(c):
            e0 = c * EC
            m0 = (c & (MDEPTH - 1)) * EC
            pltpu.async_copy(rows_hbm.at[pl.ds(e0, EC)], rows_v.at[pl.ds(m0, EC)], sem_m)
            pltpu.async_copy(cols_hbm.at[pl.ds(e0, EC)], cols_v.at[pl.ds(m0, EC)], sem_m)
            pltpu.async_copy(vals_hbm.at[pl.ds(e0, EC)], vals_v.at[pl.ds(m0, EC)], sem_m)

        def wait_meta():
            pltpu.make_async_copy(rows_hbm.at[pl.ds(0, EC)], rows_v.at[pl.ds(0, EC)], sem_m).wait()
            pltpu.make_async_copy(cols_hbm.at[pl.ds(0, EC)], cols_v.at[pl.ds(0, EC)], sem_m).wait()
            pltpu.make_async_copy(vals_hbm.at[pl.ds(0, EC)], vals_v.at[pl.ds(0, EC)], sem_m).wait()

        def fire_gather(c):
            m0 = (c & (MDEPTH - 1)) * EC
            g0 = (c & 1) * EC
            pltpu.async_copy(v_hbm.at[cols_v.at[pl.ds(m0, EC)]],
                             gbuf.at[pl.ds(g0, EC)], sem_g)

        def wait_gather():
            pltpu.make_async_copy(v_hbm.at[pl.ds(0, EC)], gbuf.at[pl.ds(0, EC)], sem_g).wait()

        @pl.when(c0 < c1)
        def _():
            fire_meta(c0)

            @pl.when(c0 + 1 < c1)
            def _():
                fire_meta(c0 + 1)

            wait_meta()
            fire_gather(c0)

        def chunk(c, carry2):
            wait_gather()

            @pl.when(c + 2 < c1)
            def _():
                fire_meta(c + 2)

            @pl.when(c + 1 < c1)
            def _():
                wait_meta()
                fire_gather(c + 1)

            m0 = (c & (MDEPTH - 1)) * EC
            g0 = (c & 1) * EC

            def group(g, carry3):
                rvec = rows_v[pl.ds(m0 + g * NLANE, NLANE)] - sub_base
                ok = (rvec >= 0) & (rvec < SUB)
                rloc = jnp.where(ok, rvec, SUB)
                vvec = vals_v[pl.ds(m0 + g * NLANE, NLANE)]
                rows_j = [rloc[j] for j in range(NLANE)]
                vals_j = [vvec[j] for j in range(NLANE)]
                for k in range(F // 32):
                    ks = pl.ds(k * 32, 32)
                    fs0 = pl.ds(k * 32, NLANE)
                    fs1 = pl.ds(k * 32 + NLANE, NLANE)
                    xs = [gbuf[g0 + g * NLANE + j, ks] for j in range(NLANE)]
                    abs_ = [plsc.unpack(x, format=plsc.PackFormat.INTERLEAVED)
                            for x in xs]
                    ys = [(vals_j[j] * abs_[j][0], vals_j[j] * abs_[j][1])
                          for j in range(NLANE)]
                    for j in range(NLANE):
                        plsc.addupdate(acc.at[rows_j[j], fs0], ys[j][0])
                        plsc.addupdate(acc.at[rows_j[j], fs1], ys[j][1])
                return carry3

            lax.fori_loop(0, EC // NLANE, group, 0)
            return carry2

        lax.fori_loop(c0, c1, chunk, 0)
        pltpu.sync_copy(acc.at[pl.ds(0, SUB)], out_hbm.at[pl.ds(sub_base, SUB)])
        return carry

    lax.fori_loop(0, NSUB, sub_block, 0)


def _spmm_sc(v, vals_p, cols_p, rows_p, bounds):
    mesh = plsc.VectorSubcoreMesh(core_axis_name="c", subcore_axis_name="s")
    fn = pl.kernel(
        _spmm_sc_body,
        mesh=mesh,
        compiler_params=pltpu.CompilerParams(use_tc_tiling_on_sc=False,
                                             needs_layout_passes=False),
        out_type=jax.ShapeDtypeStruct((M, F), jnp.float32),
        scratch_types=[
            pltpu.VMEM((NLANE,), jnp.int32),
            pltpu.VMEM((MDEPTH * EC,), jnp.int32),
            pltpu.VMEM((MDEPTH * EC,), jnp.int32),
            pltpu.VMEM((MDEPTH * EC,), jnp.float32),
            pltpu.VMEM((2 * EC, F), jnp.bfloat16),
            pltpu.VMEM((SUB + 1, F), jnp.float32),
            pltpu.SemaphoreType.DMA,
            pltpu.SemaphoreType.DMA,
        ],
    )
    return fn(v, vals_p, cols_p, rows_p, bounds)


def _gemm_body(x0_ref, x1_ref, y2_ref, w_ref, b_ref, o_ref):
    xcat = jnp.concatenate([x0_ref[...], x1_ref[...], y2_ref[...]], axis=1)
    z = lax.dot_general(
        xcat, w_ref[...], (((1,), (0,)), ((), ())),
        preferred_element_type=jnp.float32,
        precision=lax.Precision.HIGHEST,
    ) + b_ref[...]
    o_ref[...] = jnp.where(z > 0, z, jnp.exp(jnp.minimum(z, 0.0)) - 1.0)


def _gemm_elu(x0, x1, y2, wbd, bias_t):
    return pl.pallas_call(
        _gemm_body,
        grid=(M // BM,),
        in_specs=[
            pl.BlockSpec((BM, F), lambda i: (i, 0)),
            pl.BlockSpec((BM, F), lambda i: (i, 0)),
            pl.BlockSpec((BM, F), lambda i: (i, 0)),
            pl.BlockSpec((3 * F, N * OUT), lambda i: (0, 0)),
            pl.BlockSpec((1, N * OUT), lambda i: (0, 0)),
        ],
        out_specs=pl.BlockSpec((BM, N * OUT), lambda i: (i, 0)),
        out_shape=jax.ShapeDtypeStruct((M, N * OUT), jnp.float32),
    )(x0, x1, y2, wbd, bias_t)


def kernel(x, L_values, L_rows, L_cols, weight, bias):
    rows = L_rows.astype(jnp.int32)
    cols = L_cols.astype(jnp.int32)

    x0 = jnp.transpose(x, (1, 0, 2)).reshape(M, F)

    pad = NNZ_PAD - NNZ
    rows_p = jnp.concatenate([rows, jnp.full((pad,), M, jnp.int32)])
    cols_p = jnp.concatenate([cols, jnp.zeros((pad,), jnp.int32)])
    vals_p = jnp.concatenate([L_values, jnp.zeros((pad,), jnp.float32)])

    bounds = jnp.searchsorted(rows, jnp.arange(NBLK + 1, dtype=jnp.int32) * SUB).astype(jnp.int32)
    widx = (jnp.arange(NW)[:, None] * NSUB + jnp.arange(8)[None, :]).clip(0, NBLK)
    bounds = jnp.concatenate([bounds[widx].reshape(-1), jnp.full((NLANE,), NNZ, jnp.int32)])

    base = jnp.arange(NLANE, dtype=jnp.int32)
    inter = jnp.stack([base, base + NLANE], axis=1).reshape(32)
    perm = (inter[None, :] + 32 * jnp.arange(F // 32, dtype=jnp.int32)[:, None]).reshape(F)

    x0p = x0.astype(jnp.bfloat16)[:, perm]
    x1 = _spmm_sc(x0p, vals_p, cols_p, rows_p, bounds)
    x1p = x1.astype(jnp.bfloat16)[:, perm]
    y2 = _spmm_sc(x1p, vals_p, cols_p, rows_p, bounds)

    w = weight.reshape(FIN, K, OUT)
    eye_n = jnp.eye(N, dtype=jnp.float32)

    def bd(wk):
        return (eye_n[:, None, :, None] * wk[None, :, None, :]).reshape(N * FIN, N * OUT)

    wbd = jnp.concatenate([bd(w[:, 0] - w[:, 2]), bd(w[:, 1]), bd(2.0 * w[:, 2])], axis=0)
    bias_t = jnp.tile(bias, (N,)).reshape(1, N * OUT)

    out = _gemm_elu(x0, x1, y2, wbd, bias_t)
    return out.reshape(M, N, OUT).transpose(1, 0, 2)


# speedup vs baseline: 6.6511x; 1.1774x over previous
import functools

import jax
import jax.numpy as jnp
from jax import lax
from jax.experimental import pallas as pl
from jax.experimental.pallas import tpu as pltpu
from jax.experimental.pallas import tpu_sc as plsc

M = 16384
NNZ = 268435
N = 4
FIN = 64
K = 3
OUT = 64
F = N * FIN
NLANE = 16

NW = 32
ROWS_PER_W = M // NW
SUB = 256
NSUB = ROWS_PER_W // SUB
NBLK = M // SUB
EC = 128
MDEPTH = 4
NNZ_PAD = ((NNZ + EC - 1) // EC) * EC
BOUNDS_PAD = NBLK + NLANE

BM = 1024


def _spmm_sc_body(emit16, v_hbm, vals_hbm, cols_hbm, rows_hbm, bounds_hbm,
                  *refs):
    if emit16:
        out_hbm, out16_hbm, bounds_v, rows_v, cols_v, vals_v, gbuf, acc, sem_m, sem_g = refs
    else:
        out_hbm, bounds_v, rows_v, cols_v, vals_v, gbuf, acc, sem_m, sem_g = refs
    wid = lax.axis_index("s") * 2 + lax.axis_index("c")

    pltpu.sync_copy(bounds_hbm.at[pl.ds(wid * 8, NLANE)], bounds_v)
    bvec = bounds_v[...]

    bvals = [bvec[i] for i in range(NSUB + 1)]

    def sub_block(b, carry):
        sub_base = wid * ROWS_PER_W + b * SUB
        s_e = bvals[0]
        e_e = bvals[1]
        for i in range(1, NSUB + 1):
            if i < NSUB:
                s_e = jnp.where(b == i, bvals[i], s_e)
            e_e = jnp.where(b + 1 == i, bvals[i], e_e)
        c0 = s_e // EC
        c1 = (e_e + EC - 1) // EC

        def fire_meta(c):
            e0 = c * EC
            m0 = (c & (MDEPTH - 1)) * EC
            pltpu.async_copy(rows_hbm.at[pl.ds(e0, EC)], rows_v.at[pl.ds(m0, EC)], sem_m)
            pltpu.async_copy(cols_hbm.at[pl.ds(e0, EC)], cols_v.at[pl.ds(m0, EC)], sem_m)
            pltpu.async_copy(vals_hbm.at[pl.ds(e0, EC)], vals_v.at[pl.ds(m0, EC)], sem_m)

        def wait_meta():
            pltpu.make_async_copy(rows_hbm.at[pl.ds(0, EC)], rows_v.at[pl.ds(0, EC)], sem_m).wait()
            pltpu.make_async_copy(cols_hbm.at[pl.ds(0, EC)], cols_v.at[pl.ds(0, EC)], sem_m).wait()
            pltpu.make_async_copy(vals_hbm.at[pl.ds(0, EC)], vals_v.at[pl.ds(0, EC)], sem_m).wait()

        def fire_gather(c):
            m0 = (c & (MDEPTH - 1)) * EC
            g0 = (c & 1) * EC
            pltpu.async_copy(v_hbm.at[cols_v.at[pl.ds(m0, EC)]],
                             gbuf.at[pl.ds(g0, EC)], sem_g)

        def wait_gather():
            pltpu.make_async_copy(v_hbm.at[pl.ds(0, EC)], gbuf.at[pl.ds(0, EC)], sem_g).wait()

        @pl.when(c0 < c1)
        def _():
            fire_meta(c0)

            @pl.when(c0 + 1 < c1)
            def _():
                fire_meta(c0 + 1)

            wait_meta()
            fire_gather(c0)

        def zero_row(r, carry2):
            z = jnp.zeros((NLANE,), jnp.float32)
            for ff in range(F // NLANE):
                acc[r, pl.ds(ff * NLANE, NLANE)] = z
            return carry2

        lax.fori_loop(0, SUB, zero_row, 0)

        def chunk(c, carry2):
            wait_gather()

            @pl.when(c + 2 < c1)
            def _():
                fire_meta(c + 2)

            @pl.when(c + 1 < c1)
            def _():
                wait_meta()
                fire_gather(c + 1)

            m0 = (c & (MDEPTH - 1)) * EC
            g0 = (c & 1) * EC

            def group(g, carry3):
                rvec = rows_v[pl.ds(m0 + g * NLANE, NLANE)] - sub_base
                ok = (rvec >= 0) & (rvec < SUB)
                rloc = jnp.where(ok, rvec, SUB)
                vvec = vals_v[pl.ds(m0 + g * NLANE, NLANE)]
                rows_j = [rloc[j] for j in range(NLANE)]
                vals_j = [vvec[j] for j in range(NLANE)]
                for k in range(F // 32):
                    ks = pl.ds(k * 32, 32)
                    fs0 = pl.ds(k * 32, NLANE)
                    fs1 = pl.ds(k * 32 + NLANE, NLANE)
                    xs = [gbuf[g0 + g * NLANE + j, ks] for j in range(NLANE)]
                    abs_ = [plsc.unpack(x, format=plsc.PackFormat.INTERLEAVED)
                            for x in xs]
                    ys = [(vals_j[j] * abs_[j][0], vals_j[j] * abs_[j][1])
                          for j in range(NLANE)]
                    for j in range(NLANE):
                        plsc.addupdate(acc.at[rows_j[j], fs0], ys[j][0])
                        plsc.addupdate(acc.at[rows_j[j], fs1], ys[j][1])
                return carry3

            lax.fori_loop(0, EC // NLANE, group, 0)
            return carry2

        lax.fori_loop(c0, c1, chunk, 0)
        pltpu.sync_copy(acc.at[pl.ds(0, SUB)], out_hbm.at[pl.ds(sub_base, SUB)])
        if emit16:
            def pack_row(r, carry2):
                for k in range(F // 32):
                    a = acc[r, pl.ds(k * 32, NLANE)]
                    bb = acc[r, pl.ds(k * 32 + NLANE, NLANE)]
                    gbuf[r, pl.ds(k * 32, 32)] = plsc.pack(
                        a, bb, format=plsc.PackFormat.INTERLEAVED)
                return carry2

            lax.fori_loop(0, SUB, pack_row, 0)
            pltpu.sync_copy(gbuf.at[pl.ds(0, SUB)],
                            out16_hbm.at[pl.ds(sub_base, SUB)])
        return carry

    lax.fori_loop(0, NSUB, sub_block, 0)


def _spmm_sc(v, vals_p, cols_p, rows_p, bounds, emit16):
    mesh = plsc.VectorSubcoreMesh(core_axis_name="c", subcore_axis_name="s")
    out_type = jax.ShapeDtypeStruct((M, F), jnp.float32)
    if emit16:
        out_type = (out_type, jax.ShapeDtypeStruct((M, F), jnp.bfloat16))
    fn = pl.kernel(
        functools.partial(_spmm_sc_body, emit16),
        mesh=mesh,
        compiler_params=pltpu.CompilerParams(use_tc_tiling_on_sc=False,
                                             needs_layout_passes=False),
        out_type=out_type,
        scratch_types=[
            pltpu.VMEM((NLANE,), jnp.int32),
            pltpu.VMEM((MDEPTH * EC,), jnp.int32),
            pltpu.VMEM((MDEPTH * EC,), jnp.int32),
            pltpu.VMEM((MDEPTH * EC,), jnp.float32),
            pltpu.VMEM((2 * EC, F), jnp.bfloat16),
            pltpu.VMEM((SUB + 1, F), jnp.float32),
            pltpu.SemaphoreType.DMA,
            pltpu.SemaphoreType.DMA,
        ],
    )
    return fn(v, vals_p, cols_p, rows_p, bounds)


def _gemm_body(x0_ref, x1_ref, y2_ref, w_ref, b_ref, o_ref):
    xcat = jnp.concatenate([x0_ref[...], x1_ref[...], y2_ref[...]], axis=1)
    z = lax.dot_general(
        xcat, w_ref[...], (((1,), (0,)), ((), ())),
        preferred_element_type=jnp.float32,
        precision=lax.Precision.HIGHEST,
    ) + b_ref[...]
    z = jnp.where(z > 0, z, jnp.exp(jnp.minimum(z, 0.0)) - 1.0)
    for n in range(N):
        o_ref[n] = z[:, n * OUT:(n + 1) * OUT]


def _gemm_elu(x0, x1, y2, wbd, bias_t):
    return pl.pallas_call(
        _gemm_body,
        grid=(M // BM,),
        in_specs=[
            pl.BlockSpec((BM, F), lambda i: (i, 0)),
            pl.BlockSpec((BM, F), lambda i: (i, 0)),
            pl.BlockSpec((BM, F), lambda i: (i, 0)),
            pl.BlockSpec((3 * F, N * OUT), lambda i: (0, 0)),
            pl.BlockSpec((1, N * OUT), lambda i: (0, 0)),
        ],
        out_specs=pl.BlockSpec((N, BM, OUT), lambda i: (0, i, 0)),
        out_shape=jax.ShapeDtypeStruct((N, M, OUT), jnp.float32),
    )(x0, x1, y2, wbd, bias_t)


def kernel(x, L_values, L_rows, L_cols, weight, bias):
    rows = L_rows.astype(jnp.int32)
    cols = L_cols.astype(jnp.int32)

    x0 = jnp.transpose(x, (1, 0, 2)).reshape(M, F)

    pad = NNZ_PAD - NNZ
    rows_p = jnp.concatenate([rows, jnp.full((pad,), M, jnp.int32)])
    cols_p = jnp.concatenate([cols, jnp.zeros((pad,), jnp.int32)])
    vals_p = jnp.concatenate([L_values, jnp.zeros((pad,), jnp.float32)])

    bounds = jnp.searchsorted(rows, jnp.arange(NBLK + 1, dtype=jnp.int32) * SUB).astype(jnp.int32)
    widx = (jnp.arange(NW)[:, None] * NSUB + jnp.arange(8)[None, :]).clip(0, NBLK)
    bounds = jnp.concatenate([bounds[widx].reshape(-1), jnp.full((NLANE,), NNZ, jnp.int32)])

    base = jnp.arange(NLANE, dtype=jnp.int32)
    inter = jnp.stack([base, base + NLANE], axis=1).reshape(32)
    perm = (inter[None, :] + 32 * jnp.arange(F // 32, dtype=jnp.int32)[:, None]).reshape(F)

    x0p = x0.astype(jnp.bfloat16)[:, perm]
    x1, x1p = _spmm_sc(x0p, vals_p, cols_p, rows_p, bounds, emit16=True)
    y2 = _spmm_sc(x1p, vals_p, cols_p, rows_p, bounds, emit16=False)

    w = weight.reshape(FIN, K, OUT)
    eye_n = jnp.eye(N, dtype=jnp.float32)

    def bd(wk):
        return (eye_n[:, None, :, None] * wk[None, :, None, :]).reshape(N * FIN, N * OUT)

    wbd = jnp.concatenate([bd(w[:, 0] - w[:, 2]), bd(w[:, 1]), bd(2.0 * w[:, 2])], axis=0)
    bias_t = jnp.tile(bias, (N,)).reshape(1, N * OUT)

    return _gemm_elu(x0, x1, y2, wbd, bias_t)


# speedup vs baseline: 6.8283x; 1.0266x over previous
import functools

import jax
import jax.numpy as jnp
from jax import lax
from jax.experimental import pallas as pl
from jax.experimental.pallas import tpu as pltpu
from jax.experimental.pallas import tpu_sc as plsc

M = 16384
NNZ = 268435
N = 4
FIN = 64
K = 3
OUT = 64
F = N * FIN
NLANE = 16

NW = 32
ROWS_PER_W = M // NW
SUB = 256
NSUB = ROWS_PER_W // SUB
NBLK = M // SUB
EC = 128
MDEPTH = 4
NNZ_PAD = ((NNZ + EC - 1) // EC) * EC
BOUNDS_PAD = NBLK + NLANE

BM = 1024


def _spmm_sc_body(emit16, v_hbm, vals_hbm, cols_hbm, rows_hbm, bounds_hbm,
                  *refs):
    if emit16:
        out_hbm, out16_hbm, bounds_v, rows_v, cols_v, vals_v, gbuf, acc, sem_m, sem_g = refs
    else:
        out_hbm, bounds_v, rows_v, cols_v, vals_v, gbuf, acc, sem_m, sem_g = refs
    wid = lax.axis_index("s") * 2 + lax.axis_index("c")

    pltpu.sync_copy(bounds_hbm.at[pl.ds(wid * 8, NLANE)], bounds_v)
    bvec = bounds_v[...]

    bvals = [bvec[i] for i in range(NSUB + 1)]

    def sub_block(b, carry):
        sub_base = wid * ROWS_PER_W + b * SUB
        s_e = bvals[0]
        e_e = bvals[1]
        for i in range(1, NSUB + 1):
            if i < NSUB:
                s_e = jnp.where(b == i, bvals[i], s_e)
            e_e = jnp.where(b + 1 == i, bvals[i], e_e)
        c0 = s_e // EC
        c1 = (e_e + EC - 1) // EC

        def fire_meta(c):
            e0 = c * EC
            m0 = (c & (MDEPTH - 1)) * EC
            pltpu.async_copy(rows_hbm.at[pl.ds(e0, EC)], rows_v.at[pl.ds(m0, EC)], sem_m)
            pltpu.async_copy(cols_hbm.at[pl.ds(e0, EC)], cols_v.at[pl.ds(m0, EC)], sem_m)
            pltpu.async_copy(vals_hbm.at[pl.ds(e0, EC)], vals_v.at[pl.ds(m0, EC)], sem_m)

        def wait_meta():
            pltpu.make_async_copy(rows_hbm.at[pl.ds(0, EC)], rows_v.at[pl.ds(0, EC)], sem_m).wait()
            pltpu.make_async_copy(cols_hbm.at[pl.ds(0, EC)], cols_v.at[pl.ds(0, EC)], sem_m).wait()
            pltpu.make_async_copy(vals_hbm.at[pl.ds(0, EC)], vals_v.at[pl.ds(0, EC)], sem_m).wait()

        def fire_gather(c):
            m0 = (c & (MDEPTH - 1)) * EC
            g0 = (c & 1) * EC
            pltpu.async_copy(v_hbm.at[cols_v.at[pl.ds(m0, EC)]],
                             gbuf.at[pl.ds(g0, EC)], sem_g)

        def wait_gather():
            pltpu.make_async_copy(v_hbm.at[pl.ds(0, EC)], gbuf.at[pl.ds(0, EC)], sem_g).wait()

        @pl.when(c0 < c1)
        def _():
            fire_meta(c0)

            @pl.when(c0 + 1 < c1)
            def _():
                fire_meta(c0 + 1)

            wait_meta()
            fire_gather(c0)

        def zero_row(r, carry2):
            z = jnp.zeros((NLANE,), jnp.float32)
            for ff in range(F // NLANE):
                acc[r, pl.ds(ff * NLANE, NLANE)] = z
            return carry2

        lax.fori_loop(0, SUB, zero_row, 0)

        def chunk(c, carry2):
            wait_gather()

            @pl.when(c + 2 < c1)
            def _():
                fire_meta(c + 2)

            @pl.when(c + 1 < c1)
            def _():
                wait_meta()
                fire_gather(c + 1)

            m0 = (c & (MDEPTH - 1)) * EC
            g0 = (c & 1) * EC

            def group(g, carry3):
                rvec = rows_v[pl.ds(m0 + g * NLANE, NLANE)] - sub_base
                ok = (rvec >= 0) & (rvec < SUB)
                rloc = jnp.where(ok, rvec, SUB)
                vvec = vals_v[pl.ds(m0 + g * NLANE, NLANE)]
                rows_j = [rloc[j] for j in range(NLANE)]
                vals_j = [vvec[j] for j in range(NLANE)]
                for k in range(F // 32):
                    ks = pl.ds(k * 32, 32)
                    fs0 = pl.ds(k * 32, NLANE)
                    fs1 = pl.ds(k * 32 + NLANE, NLANE)
                    xs = [gbuf[g0 + g * NLANE + j, ks] for j in range(NLANE)]
                    abs_ = [plsc.unpack(x, format=plsc.PackFormat.INTERLEAVED)
                            for x in xs]
                    ys = [(vals_j[j] * abs_[j][0], vals_j[j] * abs_[j][1])
                          for j in range(NLANE)]
                    for j in range(NLANE):
                        plsc.addupdate(acc.at[rows_j[j], fs0], ys[j][0])
                        plsc.addupdate(acc.at[rows_j[j], fs1], ys[j][1])
                return carry3

            lax.fori_loop(0, EC // NLANE, group, 0)
            return carry2

        lax.fori_loop(c0, c1, chunk, 0)
        if emit16:
            pltpu.async_copy(acc.at[pl.ds(0, SUB)],
                             out_hbm.at[pl.ds(sub_base, SUB)], sem_m)

            def pack_row(r, carry2):
                as_ = [acc[r, pl.ds(k * 32, NLANE)] for k in range(F // 32)]
                bs_ = [acc[r, pl.ds(k * 32 + NLANE, NLANE)] for k in range(F // 32)]
                ps = [plsc.pack(as_[k], bs_[k], format=plsc.PackFormat.INTERLEAVED)
                      for k in range(F // 32)]
                for k in range(F // 32):
                    gbuf[r, pl.ds(k * 32, 32)] = ps[k]
                return carry2

            lax.fori_loop(0, SUB, pack_row, 0)
            pltpu.sync_copy(gbuf.at[pl.ds(0, SUB)],
                            out16_hbm.at[pl.ds(sub_base, SUB)])
            pltpu.make_async_copy(acc.at[pl.ds(0, SUB)],
                                  out_hbm.at[pl.ds(sub_base, SUB)], sem_m).wait()
        else:
            pltpu.sync_copy(acc.at[pl.ds(0, SUB)], out_hbm.at[pl.ds(sub_base, SUB)])
        return carry

    lax.fori_loop(0, NSUB, sub_block, 0)


def _spmm_sc(v, vals_p, cols_p, rows_p, bounds, emit16):
    mesh = plsc.VectorSubcoreMesh(core_axis_name="c", subcore_axis_name="s")
    out_type = jax.ShapeDtypeStruct((M, F), jnp.float32)
    if emit16:
        out_type = (out_type, jax.ShapeDtypeStruct((M, F), jnp.bfloat16))
    fn = pl.kernel(
        functools.partial(_spmm_sc_body, emit16),
        mesh=mesh,
        compiler_params=pltpu.CompilerParams(use_tc_tiling_on_sc=False,
                                             needs_layout_passes=False),
        out_type=out_type,
        scratch_types=[
            pltpu.VMEM((NLANE,), jnp.int32),
            pltpu.VMEM((MDEPTH * EC,), jnp.int32),
            pltpu.VMEM((MDEPTH * EC,), jnp.int32),
            pltpu.VMEM((MDEPTH * EC,), jnp.float32),
            pltpu.VMEM((2 * EC, F), jnp.bfloat16),
            pltpu.VMEM((SUB + 1, F), jnp.float32),
            pltpu.SemaphoreType.DMA,
            pltpu.SemaphoreType.DMA,
        ],
    )
    return fn(v, vals_p, cols_p, rows_p, bounds)


def _gemm_body(x_ref, x1_ref, y2_ref, w_ref, b_ref, o_ref):
    x0 = jnp.concatenate([x_ref[n] for n in range(N)], axis=1)
    xcat = jnp.concatenate([x0, x1_ref[...], y2_ref[...]], axis=1)
    z = lax.dot_general(
        xcat, w_ref[...], (((1,), (0,)), ((), ())),
        preferred_element_type=jnp.float32,
        precision=lax.Precision.HIGHEST,
    ) + b_ref[...]
    z = jnp.where(z > 0, z, jnp.exp(jnp.minimum(z, 0.0)) - 1.0)
    for n in range(N):
        o_ref[n] = z[:, n * OUT:(n + 1) * OUT]


def _gemm_elu(x, x1, y2, wbd, bias_t):
    return pl.pallas_call(
        _gemm_body,
        grid=(M // BM,),
        in_specs=[
            pl.BlockSpec((N, BM, FIN), lambda i: (0, i, 0)),
            pl.BlockSpec((BM, F), lambda i: (i, 0)),
            pl.BlockSpec((BM, F), lambda i: (i, 0)),
            pl.BlockSpec((3 * F, N * OUT), lambda i: (0, 0)),
            pl.BlockSpec((1, N * OUT), lambda i: (0, 0)),
        ],
        out_specs=pl.BlockSpec((N, BM, OUT), lambda i: (0, i, 0)),
        out_shape=jax.ShapeDtypeStruct((N, M, OUT), jnp.float32),
    )(x, x1, y2, wbd, bias_t)


def kernel(x, L_values, L_rows, L_cols, weight, bias):
    rows = L_rows.astype(jnp.int32)
    cols = L_cols.astype(jnp.int32)

    pad = NNZ_PAD - NNZ
    rows_p = jnp.concatenate([rows, jnp.full((pad,), M, jnp.int32)])
    cols_p = jnp.concatenate([cols, jnp.zeros((pad,), jnp.int32)])
    vals_p = jnp.concatenate([L_values, jnp.zeros((pad,), jnp.float32)])

    bounds = jnp.searchsorted(rows, jnp.arange(NBLK + 1, dtype=jnp.int32) * SUB).astype(jnp.int32)
    widx = (jnp.arange(NW)[:, None] * NSUB + jnp.arange(8)[None, :]).clip(0, NBLK)
    bounds = jnp.concatenate([bounds[widx].reshape(-1), jnp.full((NLANE,), NNZ, jnp.int32)])

    base = jnp.arange(NLANE, dtype=jnp.int32)
    inter = jnp.stack([base, base + NLANE], axis=1).reshape(32)
    perm = (inter[None, :] + 32 * jnp.arange(F // 32, dtype=jnp.int32)[:, None]).reshape(F)

    x0p = jnp.transpose(x, (1, 0, 2)).reshape(M, F).astype(jnp.bfloat16)[:, perm]
    x1, x1p = _spmm_sc(x0p, vals_p, cols_p, rows_p, bounds, emit16=True)
    y2 = _spmm_sc(x1p, vals_p, cols_p, rows_p, bounds, emit16=False)

    w = weight.reshape(FIN, K, OUT)
    eye_n = jnp.eye(N, dtype=jnp.float32)

    def bd(wk):
        return (eye_n[:, None, :, None] * wk[None, :, None, :]).reshape(N * FIN, N * OUT)

    wbd = jnp.concatenate([bd(w[:, 0] - w[:, 2]), bd(w[:, 1]), bd(2.0 * w[:, 2])], axis=0)
    bias_t = jnp.tile(bias, (N,)).reshape(1, N * OUT)

    return _gemm_elu(x, x1, y2, wbd, bias_t)
